# baseline (speedup 1.0000x reference)
import functools

import jax
import jax.numpy as jnp
from jax import lax
from jax.experimental import pallas as pl
from jax.experimental.pallas import tpu as pltpu
from jax.experimental.pallas import tpu_sc as plsc

BATCH = 4096
FEATURE_DIM = 128
SAMPLER_NUM = 8192
NUM_CLASSES = 100000

_NC = 2
_NS = 16
_NW = _NC * _NS
_CHUNK = 128
_NCHUNKS = SAMPLER_NUM // _CHUNK
_CH_PER_W = _NCHUNKS // _NW

_BN = 1024


def _sc_gather_body(ids_hbm, weight_hbm, bias_hbm, w_out, b_out,
                    idx_v, rows_v, bval_v, sem_w, sem_b):
    wid = lax.axis_index("s") * _NC + lax.axis_index("c")
    base = wid * _CH_PER_W
    pltpu.sync_copy(ids_hbm.at[pl.ds(base, _CH_PER_W)], idx_v)
    copies = []
    for j in range(_CH_PER_W):
        copies.append(
            pltpu.async_copy(weight_hbm.at[idx_v.at[j]], rows_v.at[j], sem_w))
        copies.append(
            pltpu.async_copy(bias_hbm.at[idx_v.at[j]], bval_v.at[j], sem_b))
    for c in copies:
        c.wait()
    pltpu.sync_copy(rows_v, w_out.at[pl.ds(base, _CH_PER_W)])
    pltpu.sync_copy(bval_v, b_out.at[pl.ds(base, _CH_PER_W)])


_sc_gather = functools.partial(
    pl.kernel,
    mesh=plsc.VectorSubcoreMesh(core_axis_name="c", subcore_axis_name="s"),
    out_type=[
        jax.ShapeDtypeStruct((_NCHUNKS, _CHUNK, FEATURE_DIM), jnp.float32),
        jax.ShapeDtypeStruct((_NCHUNKS, _CHUNK), jnp.float32),
    ],
    scratch_types=[
        pltpu.VMEM((_CH_PER_W, _CHUNK), jnp.int32),
        pltpu.VMEM((_CH_PER_W, _CHUNK, FEATURE_DIM), jnp.float32),
        pltpu.VMEM((_CH_PER_W, _CHUNK), jnp.float32),
        pltpu.SemaphoreType.DMA,
        pltpu.SemaphoreType.DMA,
    ],
)(_sc_gather_body)


def _tc_h_body(x_ref, wb_ref, bb_ref, h_ref):
    h_ref[...] = (
        jnp.dot(x_ref[...], wb_ref[...], preferred_element_type=jnp.float32)
        + bb_ref[...]).astype(jnp.bfloat16)


def _tc_logits_body(h_ref, w_ref, b_ref, out_ref):
    acc = lax.dot_general(
        h_ref[...], w_ref[...].astype(jnp.bfloat16),
        (((1,), (1,)), ((), ())), preferred_element_type=jnp.float32)
    out_ref[...] = acc + b_ref[...]


def kernel(x, labels, neg_ids, W_base, b_base, weight, bias):
    ids = jnp.concatenate([labels, neg_ids], axis=0).astype(jnp.int32)
    ids = ids.reshape(_NCHUNKS, _CHUNK)
    h = pl.pallas_call(
        _tc_h_body,
        in_specs=[
            pl.BlockSpec((BATCH, FEATURE_DIM), lambda: (0, 0)),
            pl.BlockSpec((FEATURE_DIM, FEATURE_DIM), lambda: (0, 0)),
            pl.BlockSpec((1, FEATURE_DIM), lambda: (0, 0)),
        ],
        out_specs=pl.BlockSpec((BATCH, FEATURE_DIM), lambda: (0, 0)),
        out_shape=jax.ShapeDtypeStruct((BATCH, FEATURE_DIM), jnp.bfloat16),
    )(x, W_base, b_base.reshape(1, FEATURE_DIM))
    w_g, b_g = _sc_gather(ids, weight, bias)
    w2 = w_g.reshape(SAMPLER_NUM, FEATURE_DIM)
    b2 = b_g.reshape(1, SAMPLER_NUM)

    logits = pl.pallas_call(
        _tc_logits_body,
        grid=(SAMPLER_NUM // _BN,),
        in_specs=[
            pl.BlockSpec((BATCH, FEATURE_DIM), lambda j: (0, 0)),
            pl.BlockSpec((_BN, FEATURE_DIM), lambda j: (j, 0)),
            pl.BlockSpec((1, _BN), lambda j: (0, j)),
        ],
        out_specs=pl.BlockSpec((BATCH, _BN), lambda j: (0, j)),
        out_shape=jax.ShapeDtypeStruct((BATCH, SAMPLER_NUM), jnp.float32),
        compiler_params=pltpu.CompilerParams(
            dimension_semantics=("arbitrary",)),
    )(h, w2, b2)

    new_labels = jnp.arange(BATCH, dtype=jnp.int32)
    return (logits, new_labels)


# speedup vs baseline: 1.0191x; 1.0191x over previous
import functools

import jax
import jax.numpy as jnp
from jax import lax
from jax.experimental import pallas as pl
from jax.experimental.pallas import tpu as pltpu
from jax.experimental.pallas import tpu_sc as plsc

BATCH = 4096
FEATURE_DIM = 128
SAMPLER_NUM = 8192
NUM_CLASSES = 100000

_NC = 2
_NS = 16
_NW = _NC * _NS
_CHUNK = 128
_NCHUNKS = SAMPLER_NUM // _CHUNK
_CH_PER_W = _NCHUNKS // _NW

_BN = 512


def _sc_gather_body(ids_hbm, weight_hbm, bias_hbm, w_out, b_out,
                    idx_v, rows_v, bval_v, sem_w, sem_b):
    wid = lax.axis_index("s") * _NC + lax.axis_index("c")
    base = wid * _CH_PER_W
    pltpu.sync_copy(ids_hbm.at[pl.ds(base, _CH_PER_W)], idx_v)
    copies = []
    for j in range(_CH_PER_W):
        copies.append(
            pltpu.async_copy(weight_hbm.at[idx_v.at[j]], rows_v.at[j], sem_w))
        copies.append(
            pltpu.async_copy(bias_hbm.at[idx_v.at[j]], bval_v.at[j], sem_b))
    for c in copies:
        c.wait()
    pltpu.sync_copy(rows_v, w_out.at[pl.ds(base, _CH_PER_W)])
    pltpu.sync_copy(bval_v, b_out.at[pl.ds(base, _CH_PER_W)])


_sc_gather = functools.partial(
    pl.kernel,
    mesh=plsc.VectorSubcoreMesh(core_axis_name="c", subcore_axis_name="s"),
    out_type=[
        jax.ShapeDtypeStruct((_NCHUNKS, _CHUNK, FEATURE_DIM), jnp.float32),
        jax.ShapeDtypeStruct((_NCHUNKS, _CHUNK), jnp.float32),
    ],
    scratch_types=[
        pltpu.VMEM((_CH_PER_W, _CHUNK), jnp.int32),
        pltpu.VMEM((_CH_PER_W, _CHUNK, FEATURE_DIM), jnp.float32),
        pltpu.VMEM((_CH_PER_W, _CHUNK), jnp.float32),
        pltpu.SemaphoreType.DMA,
        pltpu.SemaphoreType.DMA,
    ],
)(_sc_gather_body)


def _tc_h_body(x_ref, wb_ref, bb_ref, h_ref):
    h_ref[...] = (
        jnp.dot(x_ref[...], wb_ref[...], preferred_element_type=jnp.float32)
        + bb_ref[...]).astype(jnp.bfloat16)


def _tc_logits_body(h_ref, w_ref, b_ref, out_ref):
    acc = lax.dot_general(
        h_ref[...], w_ref[...].astype(jnp.bfloat16),
        (((1,), (1,)), ((), ())), preferred_element_type=jnp.float32)
    out_ref[...] = acc + b_ref[...]


def kernel(x, labels, neg_ids, W_base, b_base, weight, bias):
    ids = jnp.concatenate([labels, neg_ids], axis=0).astype(jnp.int32)
    ids = ids.reshape(_NCHUNKS, _CHUNK)
    h = pl.pallas_call(
        _tc_h_body,
        in_specs=[
            pl.BlockSpec((BATCH, FEATURE_DIM), lambda: (0, 0)),
            pl.BlockSpec((FEATURE_DIM, FEATURE_DIM), lambda: (0, 0)),
            pl.BlockSpec((1, FEATURE_DIM), lambda: (0, 0)),
        ],
        out_specs=pl.BlockSpec((BATCH, FEATURE_DIM), lambda: (0, 0)),
        out_shape=jax.ShapeDtypeStruct((BATCH, FEATURE_DIM), jnp.bfloat16),
    )(x, W_base, b_base.reshape(1, FEATURE_DIM))
    w_g, b_g = _sc_gather(ids, weight, bias)
    w2 = w_g.reshape(SAMPLER_NUM, FEATURE_DIM)
    b2 = b_g.reshape(1, SAMPLER_NUM)

    logits = pl.pallas_call(
        _tc_logits_body,
        grid=(SAMPLER_NUM // _BN,),
        in_specs=[
            pl.BlockSpec((BATCH, FEATURE_DIM), lambda j: (0, 0)),
            pl.BlockSpec((_BN, FEATURE_DIM), lambda j: (j, 0)),
            pl.BlockSpec((1, _BN), lambda j: (0, j)),
        ],
        out_specs=pl.BlockSpec((BATCH, _BN), lambda j: (0, j)),
        out_shape=jax.ShapeDtypeStruct((BATCH, SAMPLER_NUM), jnp.float32),
        compiler_params=pltpu.CompilerParams(
            dimension_semantics=("arbitrary",)),
    )(h, w2, b2)

    new_labels = jnp.arange(BATCH, dtype=jnp.int32)
    return (logits, new_labels)
